# Initial kernel scaffold; baseline (speedup 1.0000x reference)
import functools
import math

import jax
import jax.numpy as jnp
from jax import lax
from jax.experimental import pallas as pl
from jax.experimental.pallas import tpu as pltpu
from jax.experimental.pallas import tpu_sc as plsc

NC = 2
NS = 16
NW = NC * NS
L = 16


def _qkv_body(c_ref, x_ref, wq_ref, bq_ref, wk_ref, bk_ref, wv_ref, bv_ref,
              q_ref, k_ref, v_ref):
    c = c_ref[0]
    sq = jnp.sqrt(c)
    x = x_ref[...]
    r2 = jnp.sum(x * x, axis=1, keepdims=True)
    nrm = jnp.maximum(jnp.sqrt(r2), 1e-12)
    z = sq * nrm
    atz = 0.5 * jnp.log((1.0 + z) / (1.0 - z))
    t = ((2.0 / sq) * atz / nrm) * x
    dot = functools.partial(jnp.dot, preferred_element_type=jnp.float32,
                            precision=lax.Precision.HIGHEST)
    q_ref[...] = dot(t, wq_ref[...]) + bq_ref[...]
    k_ref[...] = dot(t, wk_ref[...]) + bk_ref[...]
    v_ref[...] = dot(t, wv_ref[...]) + bv_ref[...]


def _qkv(x, curvature, wqt, bq, wkt, bk, wvt, bv):
    n, d = x.shape
    blk = 2000
    grid = (n // blk,)
    row_spec = pl.BlockSpec((blk, d), lambda i: (i, 0))
    w_spec = pl.BlockSpec((d, d), lambda i: (0, 0))
    b_spec = pl.BlockSpec((1, d), lambda i: (0, 0))
    out = jax.ShapeDtypeStruct((n, d), jnp.float32)
    return pl.pallas_call(
        _qkv_body,
        grid=grid,
        in_specs=[
            pl.BlockSpec(memory_space=pltpu.SMEM),
            row_spec, w_spec, b_spec, w_spec, b_spec, w_spec, b_spec,
        ],
        out_specs=[row_spec, row_spec, row_spec],
        out_shape=[out, out, out],
    )(curvature, x, wqt, bq, wkt, bk, wvt, bv)


def _scores_body(n, e, d, chunk, kt, qt, srch, dsth, exh, dph,
                 srcv, dstv, krows, qrows, ev, stage, dsp, sem1, sem2):
    cid = lax.axis_index("c")
    sid = lax.axis_index("s")
    wid = cid * NS + sid
    epw = e // NW
    nch = epw // chunk
    inv_scale = 1.0 / math.sqrt(d)
    lanes = lax.iota(jnp.int32, L)

    @pl.when(sid == 0)
    def _():
        zero = jnp.zeros((L,), jnp.float32)

        def zb(i, carry):
            stage[pl.ds(i * L, L)] = zero
            return carry

        lax.fori_loop(0, n // L, zb, 0, unroll=8)
        pltpu.sync_copy(stage, dsp)

    plsc.subcore_barrier()

    base = wid * epw

    def chunk_body(ci, carry):
        off = base + ci * chunk
        pltpu.sync_copy(srch.at[pl.ds(off, chunk)], srcv)
        pltpu.sync_copy(dsth.at[pl.ds(off, chunk)], dstv)
        cp1 = pltpu.async_copy(kt.at[srcv], krows, sem1)
        cp2 = pltpu.async_copy(qt.at[dstv], qrows, sem2)
        cp1.wait()
        cp2.wait()
        for g in range(chunk // L):
            sv = jnp.zeros((L,), jnp.float32)
            for jj in range(L):
                j = g * L + jj
                acc = krows[j, pl.ds(0, L)] * qrows[j, pl.ds(0, L)]
                for t in range(1, d // L):
                    acc = acc + krows[j, pl.ds(t * L, L)] * qrows[j, pl.ds(t * L, L)]
                s = jnp.sum(acc)
                sv = jnp.where(lanes == jj, s, sv)
            ev[pl.ds(g * L, L)] = jnp.exp(sv * inv_scale)
        pltpu.sync_copy(ev, exh.at[pl.ds(off, chunk)])
        pltpu.sync_copy(ev, dsp.at[dstv], add=True)
        return carry

    lax.fori_loop(0, nch, chunk_body, 0)

    plsc.subcore_barrier()

    @pl.when(sid == 0)
    def _():
        pltpu.sync_copy(dsp, stage)
        pltpu.sync_copy(stage, dph.at[pl.ds(cid * n, n)])


def _edge_scores(k, q, src, dst):
    n, d = k.shape
    e = src.shape[0]
    chunk = 80
    mesh = plsc.VectorSubcoreMesh(core_axis_name="c", subcore_axis_name="s")
    fn = pl.kernel(
        functools.partial(_scores_body, n, e, d, chunk),
        out_type=(jax.ShapeDtypeStruct((e,), jnp.float32),
                  jax.ShapeDtypeStruct((NC * n,), jnp.float32)),
        mesh=mesh,
        scratch_types=[
            pltpu.VMEM((chunk,), jnp.int32),
            pltpu.VMEM((chunk,), jnp.int32),
            pltpu.VMEM((chunk, d), jnp.float32),
            pltpu.VMEM((chunk, d), jnp.float32),
            pltpu.VMEM((chunk,), jnp.float32),
            pltpu.VMEM((n,), jnp.float32),
            pltpu.VMEM_SHARED((n,), jnp.float32),
            pltpu.SemaphoreType.DMA,
            pltpu.SemaphoreType.DMA,
        ],
    )
    return fn(k, q, src, dst)


def _agg_body(n, e, d, chunk, vt, srch, dsth, exh, dph, hph,
              srcv, dstv, evb, albuf, vrows, wv, invd, tmp, stage2d, hsp, sem1):
    cid = lax.axis_index("c")
    sid = lax.axis_index("s")
    wid = cid * NS + sid
    epw = e // NW
    nch = epw // chunk
    rblk = chunk
    nrb = n // rblk
    nrb_per_tile = (nrb + NS - 1) // NS

    pltpu.sync_copy(dph.at[pl.ds(0, n)], invd)
    pltpu.sync_copy(dph.at[pl.ds(n, n)], tmp)

    def cb(i, carry):
        s = pl.ds(i * L, L)
        invd[s] = 1.0 / jnp.maximum(invd[s] + tmp[s], 1e-12)
        return carry

    lax.fori_loop(0, n // L, cb, 0, unroll=8)

    zero = jnp.zeros((L,), jnp.float32)

    def zb(i, carry):
        for t in range(d // L):
            stage2d[i, pl.ds(t * L, L)] = zero
        return carry

    lax.fori_loop(0, rblk, zb, 0, unroll=4)

    def zh(i, carry):
        c = i * NS + sid

        @pl.when(c < nrb)
        def _():
            pltpu.sync_copy(stage2d, hsp.at[pl.ds(c * rblk, rblk)])

        return carry

    lax.fori_loop(0, nrb_per_tile, zh, 0)
    plsc.subcore_barrier()

    base = wid * epw

    def chunk_body(ci, carry):
        off = base + ci * chunk
        pltpu.sync_copy(srch.at[pl.ds(off, chunk)], srcv)
        pltpu.sync_copy(dsth.at[pl.ds(off, chunk)], dstv)
        pltpu.sync_copy(exh.at[pl.ds(off, chunk)], evb)
        pltpu.async_copy(vt.at[srcv], vrows, sem1).wait()
        for g in range(chunk // L):
            di = dstv[pl.ds(g * L, L)]
            inv = plsc.load_gather(invd, [di])
            albuf[pl.ds(g * L, L)] = evb[pl.ds(g * L, L)] * inv
        for j in range(chunk):
            a = albuf[j]
            for t in range(d // L):
                wv[j, pl.ds(t * L, L)] = vrows[j, pl.ds(t * L, L)] * a
        pltpu.async_copy(wv, hsp.at[dstv], sem1, add=True).wait()
        return carry

    lax.fori_loop(0, nch, chunk_body, 0)

    plsc.subcore_barrier()

    def wb(i, carry):
        c = i * NS + sid

        @pl.when(c < nrb)
        def _():
            pltpu.sync_copy(hsp.at[pl.ds(c * rblk, rblk)], stage2d)
            pltpu.sync_copy(stage2d, hph.at[pl.ds(cid * n + c * rblk, rblk)])

        return carry

    lax.fori_loop(0, nrb_per_tile, wb, 0)


def _edge_aggregate(v, src, dst, ex, denom_p):
    n, d = v.shape
    e = src.shape[0]
    chunk = 80
    mesh = plsc.VectorSubcoreMesh(core_axis_name="c", subcore_axis_name="s")
    fn = pl.kernel(
        functools.partial(_agg_body, n, e, d, chunk),
        out_type=jax.ShapeDtypeStruct((NC * n, d), jnp.float32),
        mesh=mesh,
        scratch_types=[
            pltpu.VMEM((chunk,), jnp.int32),
            pltpu.VMEM((chunk,), jnp.int32),
            pltpu.VMEM((chunk,), jnp.float32),
            pltpu.VMEM((chunk,), jnp.float32),
            pltpu.VMEM((chunk, d), jnp.float32),
            pltpu.VMEM((chunk, d), jnp.float32),
            pltpu.VMEM((n,), jnp.float32),
            pltpu.VMEM((n,), jnp.float32),
            pltpu.VMEM((chunk, d), jnp.float32),
            pltpu.VMEM_SHARED((n, d), jnp.float32),
            pltpu.SemaphoreType.DMA,
        ],
    )
    return fn(v, src, dst, ex, denom_p)


def _expmap_body(c_ref, h0_ref, h1_ref, o_ref):
    c = c_ref[0]
    sq = jnp.sqrt(c)
    h = h0_ref[...] + h1_ref[...]
    r2 = jnp.sum(h * h, axis=1, keepdims=True)
    nrm = jnp.maximum(jnp.sqrt(r2), 1e-12)
    o_ref[...] = (jnp.tanh(sq * nrm * 0.5) / (sq * nrm)) * h


def _expmap(curvature, h0, h1):
    n, d = h0.shape
    blk = 2000
    row_spec = pl.BlockSpec((blk, d), lambda i: (i, 0))
    return pl.pallas_call(
        _expmap_body,
        grid=(n // blk,),
        in_specs=[pl.BlockSpec(memory_space=pltpu.SMEM), row_spec, row_spec],
        out_specs=row_spec,
        out_shape=jax.ShapeDtypeStruct((n, d), jnp.float32),
    )(curvature, h0, h1)


def kernel(x, edge_index, curvature, Wq, bq, Wk, bk, Wv, bv):
    n, d = x.shape
    src = edge_index[0].astype(jnp.int32)
    dst = edge_index[1].astype(jnp.int32)
    q, k, v = _qkv(x, curvature,
                   Wq.T, bq.reshape(1, d),
                   Wk.T, bk.reshape(1, d),
                   Wv.T, bv.reshape(1, d))
    ex, denom_p = _edge_scores(k, q, src, dst)
    hp = _edge_aggregate(v, src, dst, ex, denom_p)
    return _expmap(curvature, hp[:n], hp[n:])


# speedup vs baseline: 5.5216x; 5.5216x over previous
import functools
import math

import jax
import jax.numpy as jnp
from jax import lax
from jax.experimental import pallas as pl
from jax.experimental.pallas import tpu as pltpu
from jax.experimental.pallas import tpu_sc as plsc

NC = 2
NS = 16
NW = NC * NS
L = 16


def _qkv_body(c_ref, x_ref, wq_ref, bq_ref, wk_ref, bk_ref, wv_ref, bv_ref,
              q_ref, k_ref, v_ref):
    c = c_ref[0]
    sq = jnp.sqrt(c)
    x = x_ref[...]
    r2 = jnp.sum(x * x, axis=1, keepdims=True)
    nrm = jnp.maximum(jnp.sqrt(r2), 1e-12)
    z = sq * nrm
    atz = 0.5 * jnp.log((1.0 + z) / (1.0 - z))
    t = ((2.0 / sq) * atz / nrm) * x
    dot = functools.partial(jnp.dot, preferred_element_type=jnp.float32,
                            precision=lax.Precision.HIGHEST)
    q_ref[...] = dot(t, wq_ref[...]) + bq_ref[...]
    k_ref[...] = dot(t, wk_ref[...]) + bk_ref[...]
    v_ref[...] = dot(t, wv_ref[...]) + bv_ref[...]


def _qkv(x, curvature, wqt, bq, wkt, bk, wvt, bv):
    n, d = x.shape
    blk = 2000
    grid = (n // blk,)
    row_spec = pl.BlockSpec((blk, d), lambda i: (i, 0))
    w_spec = pl.BlockSpec((d, d), lambda i: (0, 0))
    b_spec = pl.BlockSpec((1, d), lambda i: (0, 0))
    out = jax.ShapeDtypeStruct((n, d), jnp.float32)
    return pl.pallas_call(
        _qkv_body,
        grid=grid,
        in_specs=[
            pl.BlockSpec(memory_space=pltpu.SMEM),
            row_spec, w_spec, b_spec, w_spec, b_spec, w_spec, b_spec,
        ],
        out_specs=[row_spec, row_spec, row_spec],
        out_shape=[out, out, out],
    )(curvature, x, wqt, bq, wkt, bk, wvt, bv)


def _scores_body(n, e, d, chunk, kt, qt, srch, dsth, exh, dph,
                 srcv, dstv, krows, qrows, ev, stage, dsp, sem1, sem2):
    cid = lax.axis_index("c")
    sid = lax.axis_index("s")
    wid = cid * NS + sid
    epw = e // NW
    nch = epw // chunk
    inv_scale = 1.0 / math.sqrt(d)
    lanes = lax.iota(jnp.int32, L)

    @pl.when(sid == 0)
    def _():
        zero = jnp.zeros((L,), jnp.float32)

        def zb(i, carry):
            stage[pl.ds(i * L, L)] = zero
            return carry

        lax.fori_loop(0, n // L, zb, 0, unroll=8)
        pltpu.sync_copy(stage, dsp)

    plsc.subcore_barrier()

    base = wid * epw

    def chunk_body(ci, carry):
        off = base + ci * chunk
        pltpu.sync_copy(srch.at[pl.ds(off, chunk)], srcv)
        pltpu.sync_copy(dsth.at[pl.ds(off, chunk)], dstv)
        cp1 = pltpu.async_copy(kt.at[srcv], krows, sem1)
        cp2 = pltpu.async_copy(qt.at[dstv], qrows, sem2)
        cp1.wait()
        cp2.wait()
        for g in range(chunk // L):
            sv = jnp.zeros((L,), jnp.float32)
            for jj in range(L):
                j = g * L + jj
                acc = krows[j, pl.ds(0, L)] * qrows[j, pl.ds(0, L)]
                for t in range(1, d // L):
                    acc = acc + krows[j, pl.ds(t * L, L)] * qrows[j, pl.ds(t * L, L)]
                s = jnp.sum(acc)
                sv = jnp.where(lanes == jj, s, sv)
            ev[pl.ds(g * L, L)] = jnp.exp(sv * inv_scale)
        pltpu.sync_copy(ev, exh.at[pl.ds(off, chunk)])
        pltpu.sync_copy(ev, dsp.at[dstv], add=True)
        return carry

    lax.fori_loop(0, nch, chunk_body, 0)

    plsc.subcore_barrier()

    @pl.when(sid == 0)
    def _():
        pltpu.sync_copy(dsp, stage)
        pltpu.sync_copy(stage, dph.at[pl.ds(cid * n, n)])


def _edge_scores(k, q, src, dst):
    n, d = k.shape
    e = src.shape[0]
    chunk = 80
    mesh = plsc.VectorSubcoreMesh(core_axis_name="c", subcore_axis_name="s")
    fn = pl.kernel(
        functools.partial(_scores_body, n, e, d, chunk),
        compiler_params=pltpu.CompilerParams(needs_layout_passes=False),
        out_type=(jax.ShapeDtypeStruct((e,), jnp.float32),
                  jax.ShapeDtypeStruct((NC * n,), jnp.float32)),
        mesh=mesh,
        scratch_types=[
            pltpu.VMEM((chunk,), jnp.int32),
            pltpu.VMEM((chunk,), jnp.int32),
            pltpu.VMEM((chunk, d), jnp.float32),
            pltpu.VMEM((chunk, d), jnp.float32),
            pltpu.VMEM((chunk,), jnp.float32),
            pltpu.VMEM((n,), jnp.float32),
            pltpu.VMEM_SHARED((n,), jnp.float32),
            pltpu.SemaphoreType.DMA,
            pltpu.SemaphoreType.DMA,
        ],
    )
    return fn(k, q, src, dst)


def _agg_body(n, e, d, chunk, vt, srch, dsth, exh, dih, hph,
              srcv, dstv, evb, vrows, wv, invd, stage2d, hsp, sem1):
    cid = lax.axis_index("c")
    sid = lax.axis_index("s")
    wid = cid * NS + sid
    epw = e // NW
    nch = epw // chunk
    rblk = chunk
    nrb = n // rblk
    nrb_per_tile = (nrb + NS - 1) // NS

    pltpu.sync_copy(dih, invd)

    zero = jnp.zeros((L,), jnp.float32)

    def zb(i, carry):
        for t in range(d // L):
            stage2d[i, pl.ds(t * L, L)] = zero
        return carry

    lax.fori_loop(0, rblk, zb, 0, unroll=4)

    def zh(i, carry):
        c = i * NS + sid

        @pl.when(c < nrb)
        def _():
            pltpu.sync_copy(stage2d, hsp.at[pl.ds(c * rblk, rblk)])

        return carry

    lax.fori_loop(0, nrb_per_tile, zh, 0)
    plsc.subcore_barrier()

    base = wid * epw

    def chunk_body(ci, carry):
        off = base + ci * chunk
        pltpu.sync_copy(srch.at[pl.ds(off, chunk)], srcv)
        pltpu.sync_copy(dsth.at[pl.ds(off, chunk)], dstv)
        pltpu.sync_copy(exh.at[pl.ds(off, chunk)], evb)
        pltpu.async_copy(vt.at[srcv], vrows, sem1).wait()
        for g in range(chunk // L):
            di = dstv[pl.ds(g * L, L)]
            inv = plsc.load_gather(invd, [di])
            a16 = evb[pl.ds(g * L, L)] * inv
            for jj in range(L):
                j = g * L + jj
                a = a16[jj]
                for t in range(d // L):
                    wv[j, pl.ds(t * L, L)] = vrows[j, pl.ds(t * L, L)] * a
        pltpu.async_copy(wv, hsp.at[dstv], sem1, add=True).wait()
        return carry

    lax.fori_loop(0, nch, chunk_body, 0)

    plsc.subcore_barrier()

    def wb(i, carry):
        c = i * NS + sid

        @pl.when(c < nrb)
        def _():
            pltpu.sync_copy(hsp.at[pl.ds(c * rblk, rblk)], stage2d)
            pltpu.sync_copy(stage2d, hph.at[pl.ds(cid * n + c * rblk, rblk)])

        return carry

    lax.fori_loop(0, nrb_per_tile, wb, 0)


def _edge_aggregate(v, src, dst, ex, denom_inv):
    n, d = v.shape
    e = src.shape[0]
    chunk = 80
    mesh = plsc.VectorSubcoreMesh(core_axis_name="c", subcore_axis_name="s")
    fn = pl.kernel(
        functools.partial(_agg_body, n, e, d, chunk),
        compiler_params=pltpu.CompilerParams(needs_layout_passes=False),
        out_type=jax.ShapeDtypeStruct((NC * n, d), jnp.float32),
        mesh=mesh,
        scratch_types=[
            pltpu.VMEM((chunk,), jnp.int32),
            pltpu.VMEM((chunk,), jnp.int32),
            pltpu.VMEM((chunk,), jnp.float32),
            pltpu.VMEM((chunk, d), jnp.float32),
            pltpu.VMEM((chunk, d), jnp.float32),
            pltpu.VMEM((n,), jnp.float32),
            pltpu.VMEM((chunk, d), jnp.float32),
            pltpu.VMEM_SHARED((n, d), jnp.float32),
            pltpu.SemaphoreType.DMA,
        ],
    )
    return fn(v, src, dst, ex, denom_inv)


def _invden_body(dp_ref, o_ref):
    dsum = dp_ref[0:1, :] + dp_ref[1:2, :]
    o_ref[...] = 1.0 / jnp.maximum(dsum, 1e-12)


def _invden(denom_p, n):
    dp = denom_p.reshape(NC, n)
    out = pl.pallas_call(
        _invden_body,
        out_shape=jax.ShapeDtypeStruct((1, n), jnp.float32),
    )(dp)
    return out.reshape(n)


def _expmap_body(c_ref, h0_ref, h1_ref, o_ref):
    c = c_ref[0]
    sq = jnp.sqrt(c)
    h = h0_ref[...] + h1_ref[...]
    r2 = jnp.sum(h * h, axis=1, keepdims=True)
    nrm = jnp.maximum(jnp.sqrt(r2), 1e-12)
    o_ref[...] = (jnp.tanh(sq * nrm * 0.5) / (sq * nrm)) * h


def _expmap(curvature, h0, h1):
    n, d = h0.shape
    blk = 2000
    row_spec = pl.BlockSpec((blk, d), lambda i: (i, 0))
    return pl.pallas_call(
        _expmap_body,
        grid=(n // blk,),
        in_specs=[pl.BlockSpec(memory_space=pltpu.SMEM), row_spec, row_spec],
        out_specs=row_spec,
        out_shape=jax.ShapeDtypeStruct((n, d), jnp.float32),
    )(curvature, h0, h1)


def kernel(x, edge_index, curvature, Wq, bq, Wk, bk, Wv, bv):
    n, d = x.shape
    src = edge_index[0].astype(jnp.int32)
    dst = edge_index[1].astype(jnp.int32)
    q, k, v = _qkv(x, curvature,
                   Wq.T, bq.reshape(1, d),
                   Wk.T, bk.reshape(1, d),
                   Wv.T, bv.reshape(1, d))
    ex, denom_p = _edge_scores(k, q, src, dst)
    denom_inv = _invden(denom_p, n)
    hp = _edge_aggregate(v, src, dst, ex, denom_inv)
    return _expmap(curvature, hp[:n], hp[n:])
